# Initial kernel scaffold; baseline (speedup 1.0000x reference)
import jax
import jax.numpy as jnp
from jax.experimental import pallas as pl


def kernel(eb_input, eb_offset, W0, W1, W2, W3):
    raise NotImplementedError("write your pallas kernel here")


# speedup vs baseline: 106.0641x; 106.0641x over previous
import jax
import jax.numpy as jnp
from jax import lax
from jax.experimental import pallas as pl
from jax.experimental.pallas import tpu as pltpu
from jax.experimental.pallas import tpu_sc as plsc

NUM_EMB = 1000000
N = 819200
D = 3
NTAB = 4

NC = 2
NS = 16
NW = NC * NS
PER_W = N // NW
CHUNK = 128
NCHUNK = PER_W // CHUNK
NPAIR = NCHUNK // 2
LANES = 16


def _sc_body(idx_hbm, tab_hbm, zacc_hbm, out_hbm,
             idx_v, buf_a, buf_b, acc, sidx_v, sem_a, sem_b):
    cid = lax.axis_index("c")
    sid = lax.axis_index("s")
    wid = sid * NC + cid
    pltpu.sync_copy(idx_hbm.at[wid], idx_v)

    sid_vec = jnp.full((LANES,), sid, jnp.int32)
    for k in range(CHUNK // LANES):
        sidx_v[pl.ds(k * LANES, LANES)] = sid_vec

    @pl.when(sid == 0)
    def _():
        pltpu.sync_copy(zacc_hbm, acc)

    plsc.subcore_barrier()

    def body(g, carry):
        c0 = 2 * g
        h_a = pltpu.async_copy(tab_hbm.at[idx_v.at[c0]], buf_a, sem_a)
        h_b = pltpu.async_copy(tab_hbm.at[idx_v.at[c0 + 1]], buf_b, sem_b)
        h_a.wait()
        pltpu.sync_copy(buf_a, acc.at[sidx_v], add=True)
        h_b.wait()
        pltpu.sync_copy(buf_b, acc.at[sidx_v], add=True)
        return carry

    lax.fori_loop(0, NPAIR, body, 0)
    plsc.subcore_barrier()

    @pl.when(sid == 0)
    def _():
        pltpu.sync_copy(acc, out_hbm.at[cid])


def kernel(eb_input, eb_offset, W0, W1, W2, W3):
    del eb_offset
    tab = jnp.concatenate(
        [W0, W1, W2, W3, jnp.zeros((NUM_EMB, LANES - NTAB * D), jnp.float32)],
        axis=1)
    idx3 = eb_input.reshape(NW, NCHUNK, CHUNK)
    zacc = jnp.zeros((NS, LANES), jnp.float32)

    mesh = plsc.VectorSubcoreMesh(core_axis_name="c", subcore_axis_name="s")
    run = pl.kernel(
        _sc_body,
        out_type=jax.ShapeDtypeStruct((NC, NS, LANES), jnp.float32),
        mesh=mesh,
        scratch_types=[
            pltpu.VMEM((NCHUNK, CHUNK), jnp.int32),
            pltpu.VMEM((CHUNK, LANES), jnp.float32),
            pltpu.VMEM((CHUNK, LANES), jnp.float32),
            pltpu.VMEM_SHARED((NS, LANES), jnp.float32),
            pltpu.VMEM((CHUNK,), jnp.int32),
            pltpu.SemaphoreType.DMA,
            pltpu.SemaphoreType.DMA,
        ],
        compiler_params=pltpu.CompilerParams(needs_layout_passes=False,
                                             use_tc_tiling_on_sc=False),
    )
    partials = run(idx3, tab, zacc)
    lanes = partials.sum(axis=(0, 1))
    return lanes[:NTAB * D].reshape(NTAB, D).sum(axis=0)
